# baseline (speedup 1.0000x reference)
import functools

import jax
import jax.numpy as jnp
from jax import lax
from jax.experimental import pallas as pl
from jax.experimental.pallas import tpu as pltpu
from jax.experimental.pallas import tpu_sc as plsc

N = 10000
NP = 10112
RPW = NP // 16
E = 320000
NW = 32
CB = 128
CH = 80
EPW = CH * CB
EP = NW * EPW
BLK = 80
GRID = N // BLK
G = 16


def _elu(z):
    return jnp.where(z > 0, z, jnp.exp(z) - 1.0)


@functools.lru_cache(maxsize=None)
def _make_spmm():
    mesh = plsc.VectorSubcoreMesh(core_axis_name="c", subcore_axis_name="s")
    scratch = [
        pltpu.VMEM((CH, CB), jnp.int32),
        pltpu.VMEM((CH, CB), jnp.int32),
        pltpu.VMEM((CB, 128), jnp.float32),
        pltpu.VMEM_SHARED((NP, 128), jnp.float32),
        pltpu.SemaphoreType.DMA,
    ]

    def body(xm, src3, dst3, zf, out, srcv, dstv, rows, shared, sem):
        c = lax.axis_index("c")
        s = lax.axis_index("s")
        wid = s * 2 + c
        pltpu.sync_copy(src3.at[wid], srcv)
        pltpu.sync_copy(dst3.at[wid], dstv)
        pltpu.sync_copy(zf, shared.at[pl.ds(s * RPW, RPW)])
        plsc.subcore_barrier()

        def step(k, carry):
            pltpu.async_copy(xm.at[srcv.at[k]], rows, sem).wait()
            pltpu.sync_copy(rows, shared.at[dstv.at[k]], add=True)
            return carry

        lax.fori_loop(0, CH, step, 0)
        plsc.subcore_barrier()
        pltpu.sync_copy(shared.at[pl.ds(s * RPW, RPW)],
                        out.at[c, pl.ds(s * RPW, RPW)])

    return pl.kernel(
        body, mesh=mesh,
        out_type=jax.ShapeDtypeStruct((2, NP, 128), jnp.float32),
        scratch_types=scratch)


@functools.lru_cache(maxsize=None)
def _make_eagg():
    mesh = plsc.VectorSubcoreMesh(core_axis_name="c", subcore_axis_name="s")
    scratch = [
        pltpu.VMEM((CH, CB), jnp.int32),
        pltpu.VMEM((CB, 128), jnp.float32),
        pltpu.VMEM((CB, 128), jnp.float32),
        pltpu.VMEM_SHARED((NP, 128), jnp.float32),
        pltpu.SemaphoreType.DMA,
        pltpu.SemaphoreType.DMA,
    ]

    def body(ea4, dst3, zf, out, dstv, eA, eB, shared, semA, semB):
        c = lax.axis_index("c")
        s = lax.axis_index("s")
        wid = s * 2 + c
        pltpu.sync_copy(dst3.at[wid], dstv)
        pltpu.sync_copy(zf, shared.at[pl.ds(s * RPW, RPW)])
        plsc.subcore_barrier()

        pltpu.async_copy(ea4.at[wid, 0], eA, semA)

        def pair(j, carry):
            k0 = 2 * j
            k1 = k0 + 1
            pltpu.make_async_copy(ea4.at[wid, k0], eA, semA).wait()
            pltpu.async_copy(ea4.at[wid, k1], eB, semB)
            pltpu.sync_copy(eA, shared.at[dstv.at[k0]], add=True)
            pltpu.make_async_copy(ea4.at[wid, k1], eB, semB).wait()

            @pl.when(k0 + 2 < CH)
            def _():
                pltpu.async_copy(ea4.at[wid, k0 + 2], eA, semA)

            pltpu.sync_copy(eB, shared.at[dstv.at[k1]], add=True)
            return carry

        lax.fori_loop(0, CH // 2, pair, 0)
        plsc.subcore_barrier()
        pltpu.sync_copy(shared.at[pl.ds(s * RPW, RPW)],
                        out.at[c, pl.ds(s * RPW, RPW)])

    return pl.kernel(
        body, mesh=mesh,
        out_type=jax.ShapeDtypeStruct((2, NP, 128), jnp.float32),
        scratch_types=scratch)


def _pad128(m):
    f = m.shape[1]
    if f == 128:
        return m
    return jnp.concatenate(
        [m, jnp.zeros((m.shape[0], 128 - f), jnp.float32)], axis=1)


def _proj(x, w):
    fin, fout = w.shape

    def body(x_ref, w_ref, o_ref):
        o_ref[...] = _pad128(jnp.dot(x_ref[...], w_ref[...],
                                     preferred_element_type=jnp.float32))

    return pl.pallas_call(
        body,
        grid=(GRID,),
        in_specs=[pl.BlockSpec((BLK, fin), lambda i: (i, 0)),
                  pl.BlockSpec((fin, fout), lambda i: (0, 0))],
        out_specs=pl.BlockSpec((BLK, 128), lambda i: (i, 0)),
        out_shape=jax.ShapeDtypeStruct((N, 128), jnp.float32),
    )(x, w)


def _layer(spmm, eagg, h_in, We, Ws, Wm_next, bm, be, bs):
    fin, fout = Ws.shape
    fnext = Wm_next.shape[1]
    bm2, be2, bs2 = (b.reshape(1, fout) for b in (bm, be, bs))

    def body(sp_ref, ea_ref, h_ref, we_ref, ws_ref, wmn_ref,
             bm_ref, be_ref, bs_ref, ho_ref, xo_ref):
        sp = (sp_ref[0] + sp_ref[1])[:, :fout]
        ea = ea_ref[0] + ea_ref[1]
        z = (sp
             + jnp.dot(h_ref[...], ws_ref[...], preferred_element_type=jnp.float32)
             + jnp.dot(ea[:, :16], we_ref[...], preferred_element_type=jnp.float32)
             + ea[:, 16:17] * (bm_ref[...] + be_ref[...])
             + bs_ref[...])
        h = _elu(z)
        ho_ref[...] = h
        xo_ref[...] = _pad128(jnp.dot(h, wmn_ref[...],
                                      preferred_element_type=jnp.float32))

    return pl.pallas_call(
        body,
        grid=(GRID,),
        in_specs=[pl.BlockSpec((2, BLK, 128), lambda i: (0, i, 0)),
                  pl.BlockSpec((2, BLK, 128), lambda i: (0, i, 0)),
                  pl.BlockSpec((BLK, fin), lambda i: (i, 0)),
                  pl.BlockSpec((16, fout), lambda i: (0, 0)),
                  pl.BlockSpec((fin, fout), lambda i: (0, 0)),
                  pl.BlockSpec((fout, fnext), lambda i: (0, 0)),
                  pl.BlockSpec((1, fout), lambda i: (0, 0)),
                  pl.BlockSpec((1, fout), lambda i: (0, 0)),
                  pl.BlockSpec((1, fout), lambda i: (0, 0))],
        out_specs=[pl.BlockSpec((BLK, fout), lambda i: (i, 0)),
                   pl.BlockSpec((BLK, 128), lambda i: (i, 0))],
        out_shape=[jax.ShapeDtypeStruct((N, fout), jnp.float32),
                   jax.ShapeDtypeStruct((N, 128), jnp.float32)],
    )(spmm, eagg, h_in, We, Ws, Wm_next, bm2, be2, bs2)


def _final(spmm, eagg, h_in, We, Ws, bm, be, bs, batch3):
    fin, fout = Ws.shape
    bm2, be2, bs2 = (b.reshape(1, fout) for b in (bm, be, bs))

    def body(sp_ref, ea_ref, h_ref, we_ref, ws_ref,
             bm_ref, be_ref, bs_ref, b_ref, o_ref, acc_p, acc_c):
        i = pl.program_id(0)
        sp = (sp_ref[0] + sp_ref[1])[:, :fout]
        ea = ea_ref[0] + ea_ref[1]
        z = (sp
             + jnp.dot(h_ref[...], ws_ref[...], preferred_element_type=jnp.float32)
             + jnp.dot(ea[:, :16], we_ref[...], preferred_element_type=jnp.float32)
             + ea[:, 16:17] * (bm_ref[...] + be_ref[...])
             + bs_ref[...])
        h = _elu(z)
        onehot = (lax.broadcasted_iota(jnp.int32, (G, BLK), 0)
                  == jnp.reshape(b_ref[...], (1, BLK))).astype(jnp.float32)

        @pl.when(i == 0)
        def _():
            acc_p[...] = jnp.zeros((G, fout), jnp.float32)
            acc_c[...] = jnp.zeros((G, fout), jnp.float32)

        acc_p[...] += jnp.dot(onehot, h, preferred_element_type=jnp.float32)
        acc_c[...] += jnp.dot(onehot, jnp.ones((BLK, fout), jnp.float32),
                              preferred_element_type=jnp.float32)

        @pl.when(i == GRID - 1)
        def _():
            o_ref[...] = acc_p[...] / jnp.maximum(acc_c[...], 1.0)

    return pl.pallas_call(
        body,
        grid=(GRID,),
        in_specs=[pl.BlockSpec((2, BLK, 128), lambda i: (0, i, 0)),
                  pl.BlockSpec((2, BLK, 128), lambda i: (0, i, 0)),
                  pl.BlockSpec((BLK, fin), lambda i: (i, 0)),
                  pl.BlockSpec((16, fout), lambda i: (0, 0)),
                  pl.BlockSpec((fin, fout), lambda i: (0, 0)),
                  pl.BlockSpec((1, fout), lambda i: (0, 0)),
                  pl.BlockSpec((1, fout), lambda i: (0, 0)),
                  pl.BlockSpec((1, fout), lambda i: (0, 0)),
                  pl.BlockSpec((1, 1, BLK), lambda i: (i, 0, 0))],
        out_specs=pl.BlockSpec((G, fout), lambda i: (0, 0)),
        out_shape=jax.ShapeDtypeStruct((G, fout), jnp.float32),
        scratch_shapes=[pltpu.VMEM((G, fout), jnp.float32),
                        pltpu.VMEM((G, fout), jnp.float32)],
    )(spmm, eagg, h_in, We, Ws, bm2, be2, bs2, batch3)


def kernel(x, edge_index, edge_attr, batch,
           Wm1, bm1, We1, be1, Ws1, bs1,
           Wm2, bm2, We2, be2, Ws2, bs2,
           Wm3, bm3, We3, be3, Ws3, bs3):
    src = edge_index[0]
    dst = edge_index[1]
    pad = EP - E
    src3 = jnp.concatenate([src, jnp.zeros((pad,), jnp.int32)]
                           ).reshape(NW, CH, CB)
    dst3 = jnp.concatenate([dst, jnp.full((pad,), N, jnp.int32)]).reshape(NW, CH, CB)
    ea128 = jnp.concatenate([edge_attr,
                             jnp.ones((E, 1), jnp.float32),
                             jnp.zeros((E, 111), jnp.float32)], axis=1)
    ea4 = jnp.concatenate([ea128, jnp.zeros((pad, 128), jnp.float32)]
                          ).reshape(NW, CH, CB, 128)
    batch3 = batch.reshape(GRID, 1, BLK)
    z128 = jnp.zeros((RPW, 128), jnp.float32)

    xm1 = _proj(x, Wm1)
    eagg = _make_eagg()(ea4, dst3, z128)
    spmm1 = _make_spmm()(xm1, src3, dst3, z128)
    h1, xm2 = _layer(spmm1, eagg, x, We1, Ws1, Wm2, bm1, be1, bs1)
    spmm2 = _make_spmm()(xm2, src3, dst3, z128)
    h2, xm3 = _layer(spmm2, eagg, h1, We2, Ws2, Wm3, bm2, be2, bs2)
    spmm3 = _make_spmm()(xm3, src3, dst3, z128)
    return _final(spmm3, eagg, h2, We3, Ws3, bm3, be3, bs3, batch3)


# speedup vs baseline: 1.1910x; 1.1910x over previous
import functools

import jax
import jax.numpy as jnp
from jax import lax
from jax.experimental import pallas as pl
from jax.experimental.pallas import tpu as pltpu
from jax.experimental.pallas import tpu_sc as plsc

N = 10000
NP = 10112
RPW = NP // 16
E = 320000
NW = 32
CB = 128
CH = 80
EPW = CH * CB
EP = NW * EPW
BLK = 80
GRID = N // BLK
G = 16


def _elu(z):
    return jnp.where(z > 0, z, jnp.exp(z) - 1.0)


@functools.lru_cache(maxsize=None)
def _make_spmm():
    mesh = plsc.VectorSubcoreMesh(core_axis_name="c", subcore_axis_name="s")
    scratch = [
        pltpu.VMEM((CH, CB), jnp.int32),
        pltpu.VMEM((CH, CB), jnp.int32),
        pltpu.VMEM((CB, 128), jnp.float32),
        pltpu.VMEM_SHARED((NP, 128), jnp.float32),
        pltpu.SemaphoreType.DMA,
    ]

    def body(xm, src3, dst3, zf, out, srcv, dstv, rows, shared, sem):
        c = lax.axis_index("c")
        s = lax.axis_index("s")
        wid = s * 2 + c
        pltpu.sync_copy(src3.at[wid], srcv)
        pltpu.sync_copy(dst3.at[wid], dstv)
        pltpu.sync_copy(zf, shared.at[pl.ds(s * RPW, RPW)])
        plsc.subcore_barrier()

        def step(k, carry):
            pltpu.async_copy(xm.at[srcv.at[k]], rows, sem).wait()
            pltpu.sync_copy(rows, shared.at[dstv.at[k]], add=True)
            return carry

        lax.fori_loop(0, CH, step, 0)
        plsc.subcore_barrier()
        pltpu.sync_copy(shared.at[pl.ds(s * RPW, RPW)],
                        out.at[c, pl.ds(s * RPW, RPW)])

    return pl.kernel(
        body, mesh=mesh,
        out_type=jax.ShapeDtypeStruct((2, NP, 128), jnp.float32),
        scratch_types=scratch)


@functools.lru_cache(maxsize=None)
def _make_eagg():
    mesh = plsc.VectorSubcoreMesh(core_axis_name="c", subcore_axis_name="s")
    scratch = [
        pltpu.VMEM((CH, CB), jnp.int32),
        pltpu.VMEM((CB, 128), jnp.float32),
        pltpu.VMEM((CB, 128), jnp.float32),
        pltpu.VMEM_SHARED((NP, 128), jnp.float32),
        pltpu.SemaphoreType.DMA,
        pltpu.SemaphoreType.DMA,
    ]

    def body(ea4, dst3, zf, out, dstv, eA, eB, shared, semA, semB):
        c = lax.axis_index("c")
        s = lax.axis_index("s")
        wid = s * 2 + c
        pltpu.sync_copy(dst3.at[wid], dstv)
        pltpu.sync_copy(zf, shared.at[pl.ds(s * RPW, RPW)])
        plsc.subcore_barrier()

        pltpu.async_copy(ea4.at[wid, 0], eA, semA)

        def pair(j, carry):
            k0 = 2 * j
            k1 = k0 + 1
            pltpu.make_async_copy(ea4.at[wid, k0], eA, semA).wait()
            pltpu.async_copy(ea4.at[wid, k1], eB, semB)
            pltpu.sync_copy(eA, shared.at[dstv.at[k0]], add=True)
            pltpu.make_async_copy(ea4.at[wid, k1], eB, semB).wait()

            @pl.when(k0 + 2 < CH)
            def _():
                pltpu.async_copy(ea4.at[wid, k0 + 2], eA, semA)

            pltpu.sync_copy(eB, shared.at[dstv.at[k1]], add=True)
            return carry

        lax.fori_loop(0, CH // 2, pair, 0)
        plsc.subcore_barrier()
        pltpu.sync_copy(shared.at[pl.ds(s * RPW, RPW)],
                        out.at[c, pl.ds(s * RPW, RPW)])

    return pl.kernel(
        body, mesh=mesh,
        out_type=jax.ShapeDtypeStruct((2, NP, 128), jnp.float32),
        scratch_types=scratch)


def _pad128(m):
    f = m.shape[1]
    if f == 128:
        return m
    return jnp.concatenate(
        [m, jnp.zeros((m.shape[0], 128 - f), jnp.float32)], axis=1)


def _proj(x, w):
    fin, fout = w.shape

    def body(x_ref, w_ref, o_ref):
        o_ref[...] = _pad128(jnp.dot(x_ref[...], w_ref[...],
                                     preferred_element_type=jnp.float32))

    return pl.pallas_call(
        body,
        grid=(GRID,),
        in_specs=[pl.BlockSpec((BLK, fin), lambda i: (i, 0)),
                  pl.BlockSpec((fin, fout), lambda i: (0, 0))],
        out_specs=pl.BlockSpec((BLK, 128), lambda i: (i, 0)),
        out_shape=jax.ShapeDtypeStruct((N, 128), jnp.float32),
    )(x, w)


def _layer(spmm, eagg, h_in, We, Ws, Wm_next, bm, be, bs):
    fin, fout = Ws.shape
    fnext = Wm_next.shape[1]
    bm2, be2, bs2 = (b.reshape(1, fout) for b in (bm, be, bs))

    def body(sp_ref, ea_ref, h_ref, we_ref, ws_ref, wmn_ref,
             bm_ref, be_ref, bs_ref, ho_ref, xo_ref):
        sp = (sp_ref[0] + sp_ref[1])[:, :fout]
        ea = ea_ref[0] + ea_ref[1]
        z = (sp
             + jnp.dot(h_ref[...], ws_ref[...], preferred_element_type=jnp.float32)
             + jnp.dot(ea[:, :16], we_ref[...], preferred_element_type=jnp.float32)
             + ea[:, 16:17] * (bm_ref[...] + be_ref[...])
             + bs_ref[...])
        h = _elu(z)
        ho_ref[...] = h
        xo_ref[...] = _pad128(jnp.dot(h, wmn_ref[...],
                                      preferred_element_type=jnp.float32))

    return pl.pallas_call(
        body,
        grid=(GRID,),
        in_specs=[pl.BlockSpec((2, BLK, 128), lambda i: (0, i, 0)),
                  pl.BlockSpec((2, BLK, 128), lambda i: (0, i, 0)),
                  pl.BlockSpec((BLK, fin), lambda i: (i, 0)),
                  pl.BlockSpec((16, fout), lambda i: (0, 0)),
                  pl.BlockSpec((fin, fout), lambda i: (0, 0)),
                  pl.BlockSpec((fout, fnext), lambda i: (0, 0)),
                  pl.BlockSpec((1, fout), lambda i: (0, 0)),
                  pl.BlockSpec((1, fout), lambda i: (0, 0)),
                  pl.BlockSpec((1, fout), lambda i: (0, 0))],
        out_specs=[pl.BlockSpec((BLK, fout), lambda i: (i, 0)),
                   pl.BlockSpec((BLK, 128), lambda i: (i, 0))],
        out_shape=[jax.ShapeDtypeStruct((N, fout), jnp.float32),
                   jax.ShapeDtypeStruct((N, 128), jnp.float32)],
    )(spmm, eagg, h_in, We, Ws, Wm_next, bm2, be2, bs2)


def _final(spmm, eagg, h_in, We, Ws, bm, be, bs, batch3):
    fin, fout = Ws.shape
    bm2, be2, bs2 = (b.reshape(1, fout) for b in (bm, be, bs))

    def body(sp_ref, ea_ref, h_ref, we_ref, ws_ref,
             bm_ref, be_ref, bs_ref, b_ref, o_ref, acc_p, acc_c):
        i = pl.program_id(0)
        sp = (sp_ref[0] + sp_ref[1])[:, :fout]
        ea = ea_ref[0] + ea_ref[1]
        z = (sp
             + jnp.dot(h_ref[...], ws_ref[...], preferred_element_type=jnp.float32)
             + jnp.dot(ea[:, :16], we_ref[...], preferred_element_type=jnp.float32)
             + ea[:, 16:17] * (bm_ref[...] + be_ref[...])
             + bs_ref[...])
        h = _elu(z)
        onehot = (lax.broadcasted_iota(jnp.int32, (G, BLK), 0)
                  == jnp.reshape(b_ref[...], (1, BLK))).astype(jnp.float32)

        @pl.when(i == 0)
        def _():
            acc_p[...] = jnp.zeros((G, fout), jnp.float32)
            acc_c[...] = jnp.zeros((G, fout), jnp.float32)

        acc_p[...] += jnp.dot(onehot, h, preferred_element_type=jnp.float32)
        acc_c[...] += jnp.dot(onehot, jnp.ones((BLK, fout), jnp.float32),
                              preferred_element_type=jnp.float32)

        @pl.when(i == GRID - 1)
        def _():
            o_ref[...] = acc_p[...] / jnp.maximum(acc_c[...], 1.0)

    return pl.pallas_call(
        body,
        grid=(GRID,),
        in_specs=[pl.BlockSpec((2, BLK, 128), lambda i: (0, i, 0)),
                  pl.BlockSpec((2, BLK, 128), lambda i: (0, i, 0)),
                  pl.BlockSpec((BLK, fin), lambda i: (i, 0)),
                  pl.BlockSpec((16, fout), lambda i: (0, 0)),
                  pl.BlockSpec((fin, fout), lambda i: (0, 0)),
                  pl.BlockSpec((1, fout), lambda i: (0, 0)),
                  pl.BlockSpec((1, fout), lambda i: (0, 0)),
                  pl.BlockSpec((1, fout), lambda i: (0, 0)),
                  pl.BlockSpec((1, 1, BLK), lambda i: (i, 0, 0))],
        out_specs=pl.BlockSpec((G, fout), lambda i: (0, 0)),
        out_shape=jax.ShapeDtypeStruct((G, fout), jnp.float32),
        scratch_shapes=[pltpu.VMEM((G, fout), jnp.float32),
                        pltpu.VMEM((G, fout), jnp.float32)],
    )(spmm, eagg, h_in, We, Ws, bm2, be2, bs2, batch3)


def kernel(x, edge_index, edge_attr, batch,
           Wm1, bm1, We1, be1, Ws1, bs1,
           Wm2, bm2, We2, be2, Ws2, bs2,
           Wm3, bm3, We3, be3, Ws3, bs3):
    src = edge_index[0]
    dst = edge_index[1]
    epw_real = E // NW
    padw = EPW - epw_real
    dump = N + (jnp.arange(padw, dtype=jnp.int32) % (NP - N))
    src3 = jnp.concatenate(
        [src.reshape(NW, epw_real),
         jnp.zeros((NW, padw), jnp.int32)], axis=1).reshape(NW, CH, CB)
    dst3 = jnp.concatenate(
        [dst.reshape(NW, epw_real),
         jnp.broadcast_to(dump, (NW, padw))], axis=1).reshape(NW, CH, CB)
    ea128 = jnp.concatenate([edge_attr,
                             jnp.ones((E, 1), jnp.float32),
                             jnp.zeros((E, 111), jnp.float32)], axis=1)
    ea4 = jnp.concatenate([ea128.reshape(NW, epw_real, 128),
                           jnp.zeros((NW, padw, 128), jnp.float32)],
                          axis=1).reshape(NW, CH, CB, 128)
    batch3 = batch.reshape(GRID, 1, BLK)
    z128 = jnp.zeros((RPW, 128), jnp.float32)

    xm1 = _proj(x, Wm1)
    eagg = _make_eagg()(ea4, dst3, z128)
    spmm1 = _make_spmm()(xm1, src3, dst3, z128)
    h1, xm2 = _layer(spmm1, eagg, x, We1, Ws1, Wm2, bm1, be1, bs1)
    spmm2 = _make_spmm()(xm2, src3, dst3, z128)
    h2, xm3 = _layer(spmm2, eagg, h1, We2, Ws2, Wm3, bm2, be2, bs2)
    spmm3 = _make_spmm()(xm3, src3, dst3, z128)
    return _final(spmm3, eagg, h2, We3, Ws3, bm3, be3, bs3, batch3)


# speedup vs baseline: 1.1923x; 1.0011x over previous
import functools

import jax
import jax.numpy as jnp
from jax import lax
from jax.experimental import pallas as pl
from jax.experimental.pallas import tpu as pltpu
from jax.experimental.pallas import tpu_sc as plsc

N = 10000
NP = 10112
RPW = NP // 16
E = 320000
NW = 32
CB = 128
CH = 80
EPW = CH * CB
EP = NW * EPW
BLK = 80
GRID = N // BLK
G = 16


def _elu(z):
    return jnp.where(z > 0, z, jnp.exp(z) - 1.0)


@functools.lru_cache(maxsize=None)
def _make_spmm():
    mesh = plsc.VectorSubcoreMesh(core_axis_name="c", subcore_axis_name="s")
    scratch = [
        pltpu.VMEM((CH, CB), jnp.int32),
        pltpu.VMEM((CH, CB), jnp.int32),
        pltpu.VMEM((CB, 128), jnp.float32),
        pltpu.VMEM_SHARED((NP, 128), jnp.float32),
        pltpu.SemaphoreType.DMA,
    ]

    def body(xm, src3, dst3, zf, out, srcv, dstv, rows, shared, sem):
        c = lax.axis_index("c")
        s = lax.axis_index("s")
        wid = s * 2 + c
        pltpu.sync_copy(src3.at[wid], srcv)
        pltpu.sync_copy(dst3.at[wid], dstv)
        pltpu.sync_copy(zf, shared.at[pl.ds(s * RPW, RPW)])
        plsc.subcore_barrier()

        def step(k, carry):
            pltpu.async_copy(xm.at[srcv.at[k]], rows, sem).wait()
            pltpu.sync_copy(rows, shared.at[dstv.at[k]], add=True)
            return carry

        lax.fori_loop(0, CH, step, 0)
        plsc.subcore_barrier()
        pltpu.sync_copy(shared.at[pl.ds(s * RPW, RPW)],
                        out.at[c, pl.ds(s * RPW, RPW)])

    return pl.kernel(
        body, mesh=mesh,
        out_type=jax.ShapeDtypeStruct((2, NP, 128), jnp.float32),
        scratch_types=scratch)


@functools.lru_cache(maxsize=None)
def _make_eagg():
    mesh = plsc.VectorSubcoreMesh(core_axis_name="c", subcore_axis_name="s")
    scratch = [
        pltpu.VMEM((CH, CB), jnp.int32),
        pltpu.VMEM((CB, 128), jnp.float32),
        pltpu.VMEM((CB, 128), jnp.float32),
        pltpu.VMEM_SHARED((NP, 128), jnp.float32),
        pltpu.SemaphoreType.DMA,
        pltpu.SemaphoreType.DMA,
    ]

    def body(ea4, dst3, zf, out, dstv, eA, eB, shared, semA, semB):
        c = lax.axis_index("c")
        s = lax.axis_index("s")
        wid = s * 2 + c
        pltpu.sync_copy(dst3.at[wid], dstv)
        pltpu.sync_copy(zf, shared.at[pl.ds(s * RPW, RPW)])
        plsc.subcore_barrier()

        pltpu.async_copy(ea4.at[wid, 0], eA, semA)

        def pair(j, carry):
            k0 = 2 * j
            k1 = k0 + 1
            pltpu.make_async_copy(ea4.at[wid, k0], eA, semA).wait()
            pltpu.async_copy(ea4.at[wid, k1], eB, semB)
            pltpu.sync_copy(eA, shared.at[dstv.at[k0]], add=True)
            pltpu.make_async_copy(ea4.at[wid, k1], eB, semB).wait()

            @pl.when(k0 + 2 < CH)
            def _():
                pltpu.async_copy(ea4.at[wid, k0 + 2], eA, semA)

            pltpu.sync_copy(eB, shared.at[dstv.at[k1]], add=True)
            return carry

        lax.fori_loop(0, CH // 2, pair, 0)
        plsc.subcore_barrier()
        pltpu.sync_copy(shared.at[pl.ds(s * RPW, RPW)],
                        out.at[c, pl.ds(s * RPW, RPW)])

    return pl.kernel(
        body, mesh=mesh,
        out_type=jax.ShapeDtypeStruct((2, NP, 128), jnp.float32),
        scratch_types=scratch)


def _pad128(m):
    f = m.shape[1]
    if f == 128:
        return m
    return jnp.concatenate(
        [m, jnp.zeros((m.shape[0], 128 - f), jnp.float32)], axis=1)


def _proj(x, w):
    fin, fout = w.shape

    def body(x_ref, w_ref, o_ref):
        o_ref[...] = _pad128(jnp.dot(x_ref[...], w_ref[...],
                                     preferred_element_type=jnp.float32))

    return pl.pallas_call(
        body,
        grid=(GRID,),
        in_specs=[pl.BlockSpec((BLK, fin), lambda i: (i, 0)),
                  pl.BlockSpec((fin, fout), lambda i: (0, 0))],
        out_specs=pl.BlockSpec((BLK, 128), lambda i: (i, 0)),
        out_shape=jax.ShapeDtypeStruct((N, 128), jnp.float32),
    )(x, w)


def _layer(spmm, eagg, h_in, We, Ws, Wm_next, bm, be, bs):
    fin, fout = Ws.shape
    fnext = Wm_next.shape[1]
    bm2, be2, bs2 = (b.reshape(1, fout) for b in (bm, be, bs))

    def body(sp_ref, ea_ref, h_ref, we_ref, ws_ref, wmn_ref,
             bm_ref, be_ref, bs_ref, ho_ref, xo_ref):
        sp = (sp_ref[0] + sp_ref[1])[:, :fout]
        ea = ea_ref[0] + ea_ref[1]
        z = (sp
             + jnp.dot(h_ref[...], ws_ref[...], preferred_element_type=jnp.float32)
             + jnp.dot(ea[:, :16], we_ref[...], preferred_element_type=jnp.float32)
             + ea[:, 16:17] * (bm_ref[...] + be_ref[...])
             + bs_ref[...])
        h = _elu(z)
        ho_ref[...] = h
        xo_ref[...] = _pad128(jnp.dot(h, wmn_ref[...],
                                      preferred_element_type=jnp.float32))

    return pl.pallas_call(
        body,
        grid=(GRID,),
        in_specs=[pl.BlockSpec((2, BLK, 128), lambda i: (0, i, 0)),
                  pl.BlockSpec((2, BLK, 128), lambda i: (0, i, 0)),
                  pl.BlockSpec((BLK, fin), lambda i: (i, 0)),
                  pl.BlockSpec((16, fout), lambda i: (0, 0)),
                  pl.BlockSpec((fin, fout), lambda i: (0, 0)),
                  pl.BlockSpec((fout, fnext), lambda i: (0, 0)),
                  pl.BlockSpec((1, fout), lambda i: (0, 0)),
                  pl.BlockSpec((1, fout), lambda i: (0, 0)),
                  pl.BlockSpec((1, fout), lambda i: (0, 0))],
        out_specs=[pl.BlockSpec((BLK, fout), lambda i: (i, 0)),
                   pl.BlockSpec((BLK, 128), lambda i: (i, 0))],
        out_shape=[jax.ShapeDtypeStruct((N, fout), jnp.float32),
                   jax.ShapeDtypeStruct((N, 128), jnp.float32)],
    )(spmm, eagg, h_in, We, Ws, Wm_next, bm2, be2, bs2)


def _final(spmm, eagg, h_in, We, Ws, bm, be, bs, batch3):
    fin, fout = Ws.shape
    bm2, be2, bs2 = (b.reshape(1, fout) for b in (bm, be, bs))

    def body(sp_ref, ea_ref, h_ref, we_ref, ws_ref,
             bm_ref, be_ref, bs_ref, b_ref, o_ref, acc_p, acc_c):
        i = pl.program_id(0)
        sp = (sp_ref[0] + sp_ref[1])[:, :fout]
        ea = ea_ref[0] + ea_ref[1]
        z = (sp
             + jnp.dot(h_ref[...], ws_ref[...], preferred_element_type=jnp.float32)
             + jnp.dot(ea[:, :16], we_ref[...], preferred_element_type=jnp.float32)
             + ea[:, 16:17] * (bm_ref[...] + be_ref[...])
             + bs_ref[...])
        h = _elu(z)
        onehot = (lax.broadcasted_iota(jnp.int32, (G, BLK), 0)
                  == jnp.reshape(b_ref[...], (1, BLK))).astype(jnp.float32)

        @pl.when(i == 0)
        def _():
            acc_p[...] = jnp.zeros((G, fout), jnp.float32)
            acc_c[...] = jnp.zeros((G, fout), jnp.float32)

        acc_p[...] += jnp.dot(onehot, h, preferred_element_type=jnp.float32)
        acc_c[...] += jnp.dot(onehot, jnp.ones((BLK, fout), jnp.float32),
                              preferred_element_type=jnp.float32)

        @pl.when(i == GRID - 1)
        def _():
            o_ref[...] = acc_p[...] / jnp.maximum(acc_c[...], 1.0)

    return pl.pallas_call(
        body,
        grid=(GRID,),
        in_specs=[pl.BlockSpec((2, BLK, 128), lambda i: (0, i, 0)),
                  pl.BlockSpec((2, BLK, 128), lambda i: (0, i, 0)),
                  pl.BlockSpec((BLK, fin), lambda i: (i, 0)),
                  pl.BlockSpec((16, fout), lambda i: (0, 0)),
                  pl.BlockSpec((fin, fout), lambda i: (0, 0)),
                  pl.BlockSpec((1, fout), lambda i: (0, 0)),
                  pl.BlockSpec((1, fout), lambda i: (0, 0)),
                  pl.BlockSpec((1, fout), lambda i: (0, 0)),
                  pl.BlockSpec((1, 1, BLK), lambda i: (i, 0, 0))],
        out_specs=pl.BlockSpec((G, fout), lambda i: (0, 0)),
        out_shape=jax.ShapeDtypeStruct((G, fout), jnp.float32),
        scratch_shapes=[pltpu.VMEM((G, fout), jnp.float32),
                        pltpu.VMEM((G, fout), jnp.float32)],
    )(spmm, eagg, h_in, We, Ws, bm2, be2, bs2, batch3)


def kernel(x, edge_index, edge_attr, batch,
           Wm1, bm1, We1, be1, Ws1, bs1,
           Wm2, bm2, We2, be2, Ws2, bs2,
           Wm3, bm3, We3, be3, Ws3, bs3):
    src = edge_index[0]
    dst = edge_index[1]
    epw_real = E // NW
    padw = EPW - epw_real
    dump = N + ((jnp.arange(padw, dtype=jnp.int32)[None, :]
                 + 7 * jnp.arange(NW, dtype=jnp.int32)[:, None]) % (NP - N))
    src3 = jnp.concatenate(
        [src.reshape(NW, epw_real),
         jnp.zeros((NW, padw), jnp.int32)], axis=1).reshape(NW, CH, CB)
    dst3 = jnp.concatenate(
        [dst.reshape(NW, epw_real), dump], axis=1).reshape(NW, CH, CB)
    ea128 = jnp.concatenate([edge_attr,
                             jnp.ones((E, 1), jnp.float32),
                             jnp.zeros((E, 111), jnp.float32)], axis=1)
    ea4 = jnp.concatenate([ea128.reshape(NW, epw_real, 128),
                           jnp.zeros((NW, padw, 128), jnp.float32)],
                          axis=1).reshape(NW, CH, CB, 128)
    batch3 = batch.reshape(GRID, 1, BLK)
    z128 = jnp.zeros((RPW, 128), jnp.float32)

    xm1 = _proj(x, Wm1)
    eagg = _make_eagg()(ea4, dst3, z128)
    spmm1 = _make_spmm()(xm1, src3, dst3, z128)
    h1, xm2 = _layer(spmm1, eagg, x, We1, Ws1, Wm2, bm1, be1, bs1)
    spmm2 = _make_spmm()(xm2, src3, dst3, z128)
    h2, xm3 = _layer(spmm2, eagg, h1, We2, Ws2, Wm3, bm2, be2, bs2)
    spmm3 = _make_spmm()(xm3, src3, dst3, z128)
    return _final(spmm3, eagg, h2, We3, Ws3, bm3, be3, bs3, batch3)


# speedup vs baseline: 1.5617x; 1.3098x over previous
import functools

import jax
import jax.numpy as jnp
from jax import lax
from jax.experimental import pallas as pl
from jax.experimental.pallas import tpu as pltpu
from jax.experimental.pallas import tpu_sc as plsc

N = 10000
NP = 10112
RPW = NP // 16
E = 320000
NW = 32
CB = 128
CH = 79
EPW = CH * CB
EP = NW * EPW
BLK = 80
GRID = N // BLK
G = 16


def _elu(z):
    return jnp.where(z > 0, z, jnp.exp(z) - 1.0)


@functools.lru_cache(maxsize=None)
def _make_spmm():
    mesh = plsc.VectorSubcoreMesh(core_axis_name="c", subcore_axis_name="s")
    scratch = [
        pltpu.VMEM((CH, CB), jnp.int32),
        pltpu.VMEM((CH, CB), jnp.int32),
        pltpu.VMEM((CB, 128), jnp.float32),
        pltpu.VMEM_SHARED((NP, 128), jnp.float32),
        pltpu.SemaphoreType.DMA,
    ]

    def body(xm, src3, dst3, zf, out, srcv, dstv, rows, shared, sem):
        c = lax.axis_index("c")
        s = lax.axis_index("s")
        wid = s * 2 + c
        pltpu.sync_copy(src3.at[wid], srcv)
        pltpu.sync_copy(dst3.at[wid], dstv)
        pltpu.sync_copy(zf, shared.at[pl.ds(s * RPW, RPW)])
        plsc.subcore_barrier()

        def step(k, carry):
            pltpu.async_copy(xm.at[srcv.at[k]], rows, sem).wait()
            pltpu.sync_copy(rows, shared.at[dstv.at[k]], add=True)
            return carry

        lax.fori_loop(0, CH, step, 0)
        plsc.subcore_barrier()
        pltpu.sync_copy(shared.at[pl.ds(s * RPW, RPW)],
                        out.at[c, pl.ds(s * RPW, RPW)])

    return pl.kernel(
        body, mesh=mesh,
        out_type=jax.ShapeDtypeStruct((2, NP, 128), jnp.float32),
        scratch_types=scratch)


@functools.lru_cache(maxsize=None)
def _make_eagg():
    mesh = plsc.VectorSubcoreMesh(core_axis_name="c", subcore_axis_name="s")
    scratch = [
        pltpu.VMEM((CH, CB), jnp.int32),
        pltpu.VMEM((CB, 128), jnp.float32),
        pltpu.VMEM((CB, 128), jnp.float32),
        pltpu.VMEM_SHARED((NP, 128), jnp.float32),
        pltpu.SemaphoreType.DMA,
        pltpu.SemaphoreType.DMA,
    ]

    def body(ea4, dst3, zf, out, dstv, eA, eB, shared, semA, semB):
        c = lax.axis_index("c")
        s = lax.axis_index("s")
        wid = s * 2 + c
        pltpu.sync_copy(dst3.at[wid], dstv)
        pltpu.sync_copy(zf, shared.at[pl.ds(s * RPW, RPW)])
        plsc.subcore_barrier()

        pltpu.async_copy(ea4.at[wid, 0], eA, semA)

        def pair(j, carry):
            k0 = 2 * j
            k1 = k0 + 1
            pltpu.make_async_copy(ea4.at[wid, k0], eA, semA).wait()
            pltpu.async_copy(ea4.at[wid, k1], eB, semB)
            pltpu.sync_copy(eA, shared.at[dstv.at[k0]], add=True)
            pltpu.make_async_copy(ea4.at[wid, k1], eB, semB).wait()

            @pl.when(k0 + 2 < CH)
            def _():
                pltpu.async_copy(ea4.at[wid, k0 + 2], eA, semA)

            pltpu.sync_copy(eB, shared.at[dstv.at[k1]], add=True)
            return carry

        lax.fori_loop(0, CH // 2, pair, 0)
        if CH % 2:
            pltpu.make_async_copy(ea4.at[wid, CH - 1], eA, semA).wait()
            pltpu.sync_copy(eA, shared.at[dstv.at[CH - 1]], add=True)
        plsc.subcore_barrier()
        pltpu.sync_copy(shared.at[pl.ds(s * RPW, RPW)],
                        out.at[c, pl.ds(s * RPW, RPW)])

    return pl.kernel(
        body, mesh=mesh,
        out_type=jax.ShapeDtypeStruct((2, NP, 128), jnp.float32),
        scratch_types=scratch)


def _pad128(m):
    f = m.shape[1]
    if f == 128:
        return m
    return jnp.concatenate(
        [m, jnp.zeros((m.shape[0], 128 - f), jnp.float32)], axis=1)


def _proj(x, w):
    fin, fout = w.shape

    def body(x_ref, w_ref, o_ref):
        o_ref[...] = _pad128(jnp.dot(x_ref[...], w_ref[...],
                                     preferred_element_type=jnp.float32))

    return pl.pallas_call(
        body,
        grid=(GRID,),
        in_specs=[pl.BlockSpec((BLK, fin), lambda i: (i, 0)),
                  pl.BlockSpec((fin, fout), lambda i: (0, 0))],
        out_specs=pl.BlockSpec((BLK, 128), lambda i: (i, 0)),
        out_shape=jax.ShapeDtypeStruct((N, 128), jnp.float32),
    )(x, w)


def _layer(spmm, eagg, h_in, We, Ws, Wm_next, bm, be, bs):
    fin, fout = Ws.shape
    fnext = Wm_next.shape[1]
    bm2, be2, bs2 = (b.reshape(1, fout) for b in (bm, be, bs))

    def body(sp_ref, ea_ref, h_ref, we_ref, ws_ref, wmn_ref,
             bm_ref, be_ref, bs_ref, ho_ref, xo_ref):
        sp = (sp_ref[0] + sp_ref[1])[:, :fout]
        ea = ea_ref[0] + ea_ref[1]
        z = (sp
             + jnp.dot(h_ref[...], ws_ref[...], preferred_element_type=jnp.float32)
             + jnp.dot(ea[:, :16], we_ref[...], preferred_element_type=jnp.float32)
             + ea[:, 16:17] * (bm_ref[...] + be_ref[...])
             + bs_ref[...])
        h = _elu(z)
        ho_ref[...] = h
        xo_ref[...] = _pad128(jnp.dot(h, wmn_ref[...],
                                      preferred_element_type=jnp.float32))

    return pl.pallas_call(
        body,
        grid=(GRID,),
        in_specs=[pl.BlockSpec((2, BLK, 128), lambda i: (0, i, 0)),
                  pl.BlockSpec((2, BLK, 128), lambda i: (0, i, 0)),
                  pl.BlockSpec((BLK, fin), lambda i: (i, 0)),
                  pl.BlockSpec((16, fout), lambda i: (0, 0)),
                  pl.BlockSpec((fin, fout), lambda i: (0, 0)),
                  pl.BlockSpec((fout, fnext), lambda i: (0, 0)),
                  pl.BlockSpec((1, fout), lambda i: (0, 0)),
                  pl.BlockSpec((1, fout), lambda i: (0, 0)),
                  pl.BlockSpec((1, fout), lambda i: (0, 0))],
        out_specs=[pl.BlockSpec((BLK, fout), lambda i: (i, 0)),
                   pl.BlockSpec((BLK, 128), lambda i: (i, 0))],
        out_shape=[jax.ShapeDtypeStruct((N, fout), jnp.float32),
                   jax.ShapeDtypeStruct((N, 128), jnp.float32)],
    )(spmm, eagg, h_in, We, Ws, Wm_next, bm2, be2, bs2)


def _final(spmm, eagg, h_in, We, Ws, bm, be, bs, batch3):
    fin, fout = Ws.shape
    bm2, be2, bs2 = (b.reshape(1, fout) for b in (bm, be, bs))

    def body(sp_ref, ea_ref, h_ref, we_ref, ws_ref,
             bm_ref, be_ref, bs_ref, b_ref, o_ref, acc_p, acc_c):
        i = pl.program_id(0)
        sp = (sp_ref[0] + sp_ref[1])[:, :fout]
        ea = ea_ref[0] + ea_ref[1]
        z = (sp
             + jnp.dot(h_ref[...], ws_ref[...], preferred_element_type=jnp.float32)
             + jnp.dot(ea[:, :16], we_ref[...], preferred_element_type=jnp.float32)
             + ea[:, 16:17] * (bm_ref[...] + be_ref[...])
             + bs_ref[...])
        h = _elu(z)
        onehot = (lax.broadcasted_iota(jnp.int32, (G, BLK), 0)
                  == jnp.reshape(b_ref[...], (1, BLK))).astype(jnp.float32)

        @pl.when(i == 0)
        def _():
            acc_p[...] = jnp.zeros((G, fout), jnp.float32)
            acc_c[...] = jnp.zeros((G, fout), jnp.float32)

        acc_p[...] += jnp.dot(onehot, h, preferred_element_type=jnp.float32)
        acc_c[...] += jnp.dot(onehot, jnp.ones((BLK, fout), jnp.float32),
                              preferred_element_type=jnp.float32)

        @pl.when(i == GRID - 1)
        def _():
            o_ref[...] = acc_p[...] / jnp.maximum(acc_c[...], 1.0)

    return pl.pallas_call(
        body,
        grid=(GRID,),
        in_specs=[pl.BlockSpec((2, BLK, 128), lambda i: (0, i, 0)),
                  pl.BlockSpec((2, BLK, 128), lambda i: (0, i, 0)),
                  pl.BlockSpec((BLK, fin), lambda i: (i, 0)),
                  pl.BlockSpec((16, fout), lambda i: (0, 0)),
                  pl.BlockSpec((fin, fout), lambda i: (0, 0)),
                  pl.BlockSpec((1, fout), lambda i: (0, 0)),
                  pl.BlockSpec((1, fout), lambda i: (0, 0)),
                  pl.BlockSpec((1, fout), lambda i: (0, 0)),
                  pl.BlockSpec((1, 1, BLK), lambda i: (i, 0, 0))],
        out_specs=pl.BlockSpec((G, fout), lambda i: (0, 0)),
        out_shape=jax.ShapeDtypeStruct((G, fout), jnp.float32),
        scratch_shapes=[pltpu.VMEM((G, fout), jnp.float32),
                        pltpu.VMEM((G, fout), jnp.float32)],
    )(spmm, eagg, h_in, We, Ws, bm2, be2, bs2, batch3)


def kernel(x, edge_index, edge_attr, batch,
           Wm1, bm1, We1, be1, Ws1, bs1,
           Wm2, bm2, We2, be2, Ws2, bs2,
           Wm3, bm3, We3, be3, Ws3, bs3):
    src = edge_index[0]
    dst = edge_index[1]
    epw_real = E // NW
    padw = EPW - epw_real
    dump = N + ((jnp.arange(padw, dtype=jnp.int32)[None, :]
                 + 7 * jnp.arange(NW, dtype=jnp.int32)[:, None]) % (NP - N))
    src3 = jnp.concatenate(
        [src.reshape(NW, epw_real),
         jnp.zeros((NW, padw), jnp.int32)], axis=1).reshape(NW, CH, CB)
    dst3 = jnp.concatenate(
        [dst.reshape(NW, epw_real), dump], axis=1).reshape(NW, CH, CB)
    ea128 = jnp.concatenate([edge_attr,
                             jnp.ones((E, 1), jnp.float32),
                             jnp.zeros((E, 111), jnp.float32)], axis=1)
    ea4 = jnp.concatenate([ea128.reshape(NW, epw_real, 128),
                           jnp.zeros((NW, padw, 128), jnp.float32)],
                          axis=1).reshape(NW, CH, CB, 128)
    batch3 = batch.reshape(GRID, 1, BLK)
    z128 = jnp.zeros((RPW, 128), jnp.float32)

    xm1 = _proj(x, Wm1)
    eagg = _make_eagg()(ea4, dst3, z128)
    spmm1 = _make_spmm()(xm1, src3, dst3, z128)
    h1, xm2 = _layer(spmm1, eagg, x, We1, Ws1, Wm2, bm1, be1, bs1)
    spmm2 = _make_spmm()(xm2, src3, dst3, z128)
    h2, xm3 = _layer(spmm2, eagg, h1, We2, Ws2, Wm3, bm2, be2, bs2)
    spmm3 = _make_spmm()(xm3, src3, dst3, z128)
    return _final(spmm3, eagg, h2, We3, Ws3, bm3, be3, bs3, batch3)


# speedup vs baseline: 1.6658x; 1.0667x over previous
import functools

import jax
import jax.numpy as jnp
from jax import lax
from jax.experimental import pallas as pl
from jax.experimental.pallas import tpu as pltpu
from jax.experimental.pallas import tpu_sc as plsc

N = 10000
NP = 10112
RPW = NP // 16
E = 320000
NW = 32
CB = 128
CH = 79
EPW = CH * CB
EP = NW * EPW
BLK = 80
GRID = N // BLK
G = 16


def _elu(z):
    return jnp.where(z > 0, z, jnp.exp(z) - 1.0)


@functools.lru_cache(maxsize=None)
def _make_spmm():
    mesh = plsc.VectorSubcoreMesh(core_axis_name="c", subcore_axis_name="s")
    scratch = [
        pltpu.VMEM((CH, CB), jnp.int32),
        pltpu.VMEM((1, CB), jnp.int32),
        pltpu.VMEM((1, CB), jnp.int32),
        pltpu.VMEM((CB, 128), jnp.float32),
        pltpu.VMEM((CB, 128), jnp.float32),
        pltpu.VMEM_SHARED((NP, 128), jnp.float32),
        pltpu.SemaphoreType.DMA,
        pltpu.SemaphoreType.DMA,
        pltpu.SemaphoreType.DMA,
        pltpu.SemaphoreType.DMA,
    ]

    def body(xm, src4, dst3, zf, out, dstv, srcA, srcB, gA, gB, shared,
             semA, semB, isemA, isemB):
        c = lax.axis_index("c")
        s = lax.axis_index("s")
        wid = s * 2 + c
        pltpu.sync_copy(dst3.at[wid], dstv)
        pltpu.sync_copy(zf, shared.at[pl.ds(s * RPW, RPW)])
        plsc.subcore_barrier()

        pltpu.sync_copy(src4.at[wid, 0], srcA)
        pltpu.sync_copy(src4.at[wid, 1], srcB)
        pltpu.async_copy(xm.at[srcA.at[0]], gA, semA)

        def pair(j, carry):
            k0 = 2 * j
            k1 = k0 + 1
            pltpu.make_async_copy(xm.at[srcA.at[0]], gA, semA).wait()

            @pl.when(j > 0)
            def _():
                pltpu.make_async_copy(src4.at[wid, k1], srcB, isemB).wait()

            pltpu.async_copy(xm.at[srcB.at[0]], gB, semB)
            pltpu.sync_copy(gA, shared.at[dstv.at[k0]], add=True)

            @pl.when(k0 + 2 < CH)
            def _():
                pltpu.async_copy(src4.at[wid, k0 + 2], srcA, isemA)

            pltpu.make_async_copy(xm.at[srcB.at[0]], gB, semB).wait()

            @pl.when(k0 + 3 < CH)
            def _():
                pltpu.async_copy(src4.at[wid, k0 + 3], srcB, isemB)

            @pl.when(k0 + 2 < CH)
            def _():
                pltpu.make_async_copy(src4.at[wid, k0 + 2], srcA, isemA).wait()
                pltpu.async_copy(xm.at[srcA.at[0]], gA, semA)

            pltpu.sync_copy(gB, shared.at[dstv.at[k1]], add=True)
            return carry

        lax.fori_loop(0, CH // 2, pair, 0)
        if CH % 2:
            pltpu.make_async_copy(xm.at[srcA.at[0]], gA, semA).wait()
            pltpu.sync_copy(gA, shared.at[dstv.at[CH - 1]], add=True)
        plsc.subcore_barrier()
        pltpu.sync_copy(shared.at[pl.ds(s * RPW, RPW)],
                        out.at[c, pl.ds(s * RPW, RPW)])

    return pl.kernel(
        body, mesh=mesh,
        out_type=jax.ShapeDtypeStruct((2, NP, 128), jnp.float32),
        scratch_types=scratch)


@functools.lru_cache(maxsize=None)
def _make_eagg():
    mesh = plsc.VectorSubcoreMesh(core_axis_name="c", subcore_axis_name="s")
    scratch = [
        pltpu.VMEM((CH, CB), jnp.int32),
        pltpu.VMEM((CB, 128), jnp.float32),
        pltpu.VMEM((CB, 128), jnp.float32),
        pltpu.VMEM_SHARED((NP, 128), jnp.float32),
        pltpu.SemaphoreType.DMA,
        pltpu.SemaphoreType.DMA,
    ]

    def body(ea4, dst3, zf, out, dstv, eA, eB, shared, semA, semB):
        c = lax.axis_index("c")
        s = lax.axis_index("s")
        wid = s * 2 + c
        pltpu.sync_copy(dst3.at[wid], dstv)
        pltpu.sync_copy(zf, shared.at[pl.ds(s * RPW, RPW)])
        plsc.subcore_barrier()

        pltpu.async_copy(ea4.at[wid, 0], eA, semA)

        def pair(j, carry):
            k0 = 2 * j
            k1 = k0 + 1
            pltpu.make_async_copy(ea4.at[wid, k0], eA, semA).wait()
            pltpu.async_copy(ea4.at[wid, k1], eB, semB)
            pltpu.sync_copy(eA, shared.at[dstv.at[k0]], add=True)
            pltpu.make_async_copy(ea4.at[wid, k1], eB, semB).wait()

            @pl.when(k0 + 2 < CH)
            def _():
                pltpu.async_copy(ea4.at[wid, k0 + 2], eA, semA)

            pltpu.sync_copy(eB, shared.at[dstv.at[k1]], add=True)
            return carry

        lax.fori_loop(0, CH // 2, pair, 0)
        if CH % 2:
            pltpu.make_async_copy(ea4.at[wid, CH - 1], eA, semA).wait()
            pltpu.sync_copy(eA, shared.at[dstv.at[CH - 1]], add=True)
        plsc.subcore_barrier()
        pltpu.sync_copy(shared.at[pl.ds(s * RPW, RPW)],
                        out.at[c, pl.ds(s * RPW, RPW)])

    return pl.kernel(
        body, mesh=mesh,
        out_type=jax.ShapeDtypeStruct((2, NP, 128), jnp.float32),
        scratch_types=scratch)


def _pad128(m):
    f = m.shape[1]
    if f == 128:
        return m
    return jnp.concatenate(
        [m, jnp.zeros((m.shape[0], 128 - f), jnp.float32)], axis=1)


def _proj(x, w):
    fin, fout = w.shape

    def body(x_ref, w_ref, o_ref):
        o_ref[...] = _pad128(jnp.dot(x_ref[...], w_ref[...],
                                     preferred_element_type=jnp.float32))

    return pl.pallas_call(
        body,
        grid=(GRID,),
        in_specs=[pl.BlockSpec((BLK, fin), lambda i: (i, 0)),
                  pl.BlockSpec((fin, fout), lambda i: (0, 0))],
        out_specs=pl.BlockSpec((BLK, 128), lambda i: (i, 0)),
        out_shape=jax.ShapeDtypeStruct((N, 128), jnp.float32),
    )(x, w)


def _layer(spmm, eagg, h_in, We, Ws, Wm_next, bm, be, bs):
    fin, fout = Ws.shape
    fnext = Wm_next.shape[1]
    bm2, be2, bs2 = (b.reshape(1, fout) for b in (bm, be, bs))

    def body(sp_ref, ea_ref, h_ref, we_ref, ws_ref, wmn_ref,
             bm_ref, be_ref, bs_ref, ho_ref, xo_ref):
        sp = (sp_ref[0] + sp_ref[1])[:, :fout]
        ea = ea_ref[0] + ea_ref[1]
        z = (sp
             + jnp.dot(h_ref[...], ws_ref[...], preferred_element_type=jnp.float32)
             + jnp.dot(ea[:, :16], we_ref[...], preferred_element_type=jnp.float32)
             + ea[:, 16:17] * (bm_ref[...] + be_ref[...])
             + bs_ref[...])
        h = _elu(z)
        ho_ref[...] = h
        xo_ref[...] = _pad128(jnp.dot(h, wmn_ref[...],
                                      preferred_element_type=jnp.float32))

    return pl.pallas_call(
        body,
        grid=(GRID,),
        in_specs=[pl.BlockSpec((2, BLK, 128), lambda i: (0, i, 0)),
                  pl.BlockSpec((2, BLK, 128), lambda i: (0, i, 0)),
                  pl.BlockSpec((BLK, fin), lambda i: (i, 0)),
                  pl.BlockSpec((16, fout), lambda i: (0, 0)),
                  pl.BlockSpec((fin, fout), lambda i: (0, 0)),
                  pl.BlockSpec((fout, fnext), lambda i: (0, 0)),
                  pl.BlockSpec((1, fout), lambda i: (0, 0)),
                  pl.BlockSpec((1, fout), lambda i: (0, 0)),
                  pl.BlockSpec((1, fout), lambda i: (0, 0))],
        out_specs=[pl.BlockSpec((BLK, fout), lambda i: (i, 0)),
                   pl.BlockSpec((BLK, 128), lambda i: (i, 0))],
        out_shape=[jax.ShapeDtypeStruct((N, fout), jnp.float32),
                   jax.ShapeDtypeStruct((N, 128), jnp.float32)],
    )(spmm, eagg, h_in, We, Ws, Wm_next, bm2, be2, bs2)


def _final(spmm, eagg, h_in, We, Ws, bm, be, bs, batch3):
    fin, fout = Ws.shape
    bm2, be2, bs2 = (b.reshape(1, fout) for b in (bm, be, bs))

    def body(sp_ref, ea_ref, h_ref, we_ref, ws_ref,
             bm_ref, be_ref, bs_ref, b_ref, o_ref, acc_p, acc_c):
        i = pl.program_id(0)
        sp = (sp_ref[0] + sp_ref[1])[:, :fout]
        ea = ea_ref[0] + ea_ref[1]
        z = (sp
             + jnp.dot(h_ref[...], ws_ref[...], preferred_element_type=jnp.float32)
             + jnp.dot(ea[:, :16], we_ref[...], preferred_element_type=jnp.float32)
             + ea[:, 16:17] * (bm_ref[...] + be_ref[...])
             + bs_ref[...])
        h = _elu(z)
        onehot = (lax.broadcasted_iota(jnp.int32, (G, BLK), 0)
                  == jnp.reshape(b_ref[...], (1, BLK))).astype(jnp.float32)

        @pl.when(i == 0)
        def _():
            acc_p[...] = jnp.zeros((G, fout), jnp.float32)
            acc_c[...] = jnp.zeros((G, fout), jnp.float32)

        acc_p[...] += jnp.dot(onehot, h, preferred_element_type=jnp.float32)
        acc_c[...] += jnp.dot(onehot, jnp.ones((BLK, fout), jnp.float32),
                              preferred_element_type=jnp.float32)

        @pl.when(i == GRID - 1)
        def _():
            o_ref[...] = acc_p[...] / jnp.maximum(acc_c[...], 1.0)

    return pl.pallas_call(
        body,
        grid=(GRID,),
        in_specs=[pl.BlockSpec((2, BLK, 128), lambda i: (0, i, 0)),
                  pl.BlockSpec((2, BLK, 128), lambda i: (0, i, 0)),
                  pl.BlockSpec((BLK, fin), lambda i: (i, 0)),
                  pl.BlockSpec((16, fout), lambda i: (0, 0)),
                  pl.BlockSpec((fin, fout), lambda i: (0, 0)),
                  pl.BlockSpec((1, fout), lambda i: (0, 0)),
                  pl.BlockSpec((1, fout), lambda i: (0, 0)),
                  pl.BlockSpec((1, fout), lambda i: (0, 0)),
                  pl.BlockSpec((1, 1, BLK), lambda i: (i, 0, 0))],
        out_specs=pl.BlockSpec((G, fout), lambda i: (0, 0)),
        out_shape=jax.ShapeDtypeStruct((G, fout), jnp.float32),
        scratch_shapes=[pltpu.VMEM((G, fout), jnp.float32),
                        pltpu.VMEM((G, fout), jnp.float32)],
    )(spmm, eagg, h_in, We, Ws, bm2, be2, bs2, batch3)


def kernel(x, edge_index, edge_attr, batch,
           Wm1, bm1, We1, be1, Ws1, bs1,
           Wm2, bm2, We2, be2, Ws2, bs2,
           Wm3, bm3, We3, be3, Ws3, bs3):
    src = edge_index[0]
    dst = edge_index[1]
    epw_real = E // NW
    padw = EPW - epw_real
    dump = N + ((jnp.arange(padw, dtype=jnp.int32)[None, :]
                 + 7 * jnp.arange(NW, dtype=jnp.int32)[:, None]) % (NP - N))
    src4 = jnp.concatenate(
        [src.reshape(NW, epw_real),
         jnp.zeros((NW, padw), jnp.int32)], axis=1).reshape(NW, CH, 1, CB)
    dst3 = jnp.concatenate(
        [dst.reshape(NW, epw_real), dump], axis=1).reshape(NW, CH, CB)
    ea128 = jnp.concatenate([edge_attr,
                             jnp.ones((E, 1), jnp.float32),
                             jnp.zeros((E, 111), jnp.float32)], axis=1)
    ea4 = jnp.concatenate([ea128.reshape(NW, epw_real, 128),
                           jnp.zeros((NW, padw, 128), jnp.float32)],
                          axis=1).reshape(NW, CH, CB, 128)
    batch3 = batch.reshape(GRID, 1, BLK)
    z128 = jnp.zeros((RPW, 128), jnp.float32)

    xm1 = _proj(x, Wm1)
    eagg = _make_eagg()(ea4, dst3, z128)
    spmm1 = _make_spmm()(xm1, src4, dst3, z128)
    h1, xm2 = _layer(spmm1, eagg, x, We1, Ws1, Wm2, bm1, be1, bs1)
    spmm2 = _make_spmm()(xm2, src4, dst3, z128)
    h2, xm3 = _layer(spmm2, eagg, h1, We2, Ws2, Wm3, bm2, be2, bs2)
    spmm3 = _make_spmm()(xm3, src4, dst3, z128)
    return _final(spmm3, eagg, h2, We3, Ws3, bm3, be3, bs3, batch3)


# speedup vs baseline: 1.6673x; 1.0009x over previous
import functools

import jax
import jax.numpy as jnp
from jax import lax
from jax.experimental import pallas as pl
from jax.experimental.pallas import tpu as pltpu
from jax.experimental.pallas import tpu_sc as plsc

N = 10000
NP = 10112
RPW = NP // 16
E = 320000
NW = 32
CB = 128
CH = 79
EPW = CH * CB
EP = NW * EPW
BLK = 80
GRID = N // BLK
G = 16


def _elu(z):
    return jnp.where(z > 0, z, jnp.exp(z) - 1.0)


@functools.lru_cache(maxsize=None)
def _make_spmm():
    mesh = plsc.VectorSubcoreMesh(core_axis_name="c", subcore_axis_name="s")
    scratch = [
        pltpu.VMEM((CH, CB), jnp.int32),
        pltpu.VMEM((1, CB), jnp.int32),
        pltpu.VMEM((1, CB), jnp.int32),
        pltpu.VMEM((CB, 128), jnp.float32),
        pltpu.VMEM((CB, 128), jnp.float32),
        pltpu.VMEM_SHARED((NP, 128), jnp.float32),
        pltpu.SemaphoreType.DMA,
        pltpu.SemaphoreType.DMA,
        pltpu.SemaphoreType.DMA,
        pltpu.SemaphoreType.DMA,
        pltpu.SemaphoreType.DMA,
        pltpu.SemaphoreType.DMA,
    ]

    def body(xm, src4, dst3, zf, out, dstv, srcA, srcB, gA, gB, shared,
             semA, semB, isemA, isemB, ssemA, ssemB):
        c = lax.axis_index("c")
        s = lax.axis_index("s")
        wid = s * 2 + c
        pltpu.sync_copy(dst3.at[wid], dstv)
        pltpu.sync_copy(zf, shared.at[pl.ds(s * RPW, RPW)])
        plsc.subcore_barrier()

        pltpu.sync_copy(src4.at[wid, 0], srcA)
        pltpu.sync_copy(src4.at[wid, 1], srcB)
        pltpu.async_copy(xm.at[srcA.at[0]], gA, semA)

        def pair(j, carry):
            k0 = 2 * j
            k1 = k0 + 1
            pltpu.make_async_copy(xm.at[srcA.at[0]], gA, semA).wait()

            @pl.when(j > 0)
            def _():
                pltpu.make_async_copy(gB, shared.at[dstv.at[k0]], ssemB).wait()
                pltpu.make_async_copy(src4.at[wid, k1], srcB, isemB).wait()

            pltpu.async_copy(xm.at[srcB.at[0]], gB, semB)
            pltpu.async_copy(gA, shared.at[dstv.at[k0]], ssemA, add=True)

            @pl.when(k0 + 2 < CH)
            def _():
                pltpu.async_copy(src4.at[wid, k0 + 2], srcA, isemA)

            pltpu.make_async_copy(xm.at[srcB.at[0]], gB, semB).wait()

            @pl.when(k0 + 3 < CH)
            def _():
                pltpu.async_copy(src4.at[wid, k0 + 3], srcB, isemB)

            pltpu.async_copy(gB, shared.at[dstv.at[k1]], ssemB, add=True)

            @pl.when(k0 + 2 < CH)
            def _():
                pltpu.make_async_copy(gA, shared.at[dstv.at[k0]], ssemA).wait()
                pltpu.make_async_copy(src4.at[wid, k0 + 2], srcA, isemA).wait()
                pltpu.async_copy(xm.at[srcA.at[0]], gA, semA)

            return carry

        lax.fori_loop(0, CH // 2, pair, 0)
        if CH % 2:
            pltpu.make_async_copy(xm.at[srcA.at[0]], gA, semA).wait()
            pltpu.sync_copy(gA, shared.at[dstv.at[CH - 1]], add=True)
        pltpu.make_async_copy(gB, shared.at[dstv.at[CH - 2]], ssemB).wait()
        plsc.subcore_barrier()
        pltpu.sync_copy(shared.at[pl.ds(s * RPW, RPW)],
                        out.at[c, pl.ds(s * RPW, RPW)])

    return pl.kernel(
        body, mesh=mesh,
        out_type=jax.ShapeDtypeStruct((2, NP, 128), jnp.float32),
        scratch_types=scratch)


@functools.lru_cache(maxsize=None)
def _make_eagg():
    mesh = plsc.VectorSubcoreMesh(core_axis_name="c", subcore_axis_name="s")
    scratch = [
        pltpu.VMEM((CH, CB), jnp.int32),
        pltpu.VMEM((CB, 128), jnp.float32),
        pltpu.VMEM((CB, 128), jnp.float32),
        pltpu.VMEM_SHARED((NP, 128), jnp.float32),
        pltpu.SemaphoreType.DMA,
        pltpu.SemaphoreType.DMA,
    ]

    def body(ea4, dst3, zf, out, dstv, eA, eB, shared, semA, semB):
        c = lax.axis_index("c")
        s = lax.axis_index("s")
        wid = s * 2 + c
        pltpu.sync_copy(dst3.at[wid], dstv)
        pltpu.sync_copy(zf, shared.at[pl.ds(s * RPW, RPW)])
        plsc.subcore_barrier()

        pltpu.async_copy(ea4.at[wid, 0], eA, semA)

        def pair(j, carry):
            k0 = 2 * j
            k1 = k0 + 1
            pltpu.make_async_copy(ea4.at[wid, k0], eA, semA).wait()
            pltpu.async_copy(ea4.at[wid, k1], eB, semB)
            pltpu.sync_copy(eA, shared.at[dstv.at[k0]], add=True)
            pltpu.make_async_copy(ea4.at[wid, k1], eB, semB).wait()

            @pl.when(k0 + 2 < CH)
            def _():
                pltpu.async_copy(ea4.at[wid, k0 + 2], eA, semA)

            pltpu.sync_copy(eB, shared.at[dstv.at[k1]], add=True)
            return carry

        lax.fori_loop(0, CH // 2, pair, 0)
        if CH % 2:
            pltpu.make_async_copy(ea4.at[wid, CH - 1], eA, semA).wait()
            pltpu.sync_copy(eA, shared.at[dstv.at[CH - 1]], add=True)
        plsc.subcore_barrier()
        pltpu.sync_copy(shared.at[pl.ds(s * RPW, RPW)],
                        out.at[c, pl.ds(s * RPW, RPW)])

    return pl.kernel(
        body, mesh=mesh,
        out_type=jax.ShapeDtypeStruct((2, NP, 128), jnp.float32),
        scratch_types=scratch)


def _pad128(m):
    f = m.shape[1]
    if f == 128:
        return m
    return jnp.concatenate(
        [m, jnp.zeros((m.shape[0], 128 - f), jnp.float32)], axis=1)


def _proj(x, w):
    fin, fout = w.shape

    def body(x_ref, w_ref, o_ref):
        o_ref[...] = _pad128(jnp.dot(x_ref[...], w_ref[...],
                                     preferred_element_type=jnp.float32))

    return pl.pallas_call(
        body,
        grid=(GRID,),
        in_specs=[pl.BlockSpec((BLK, fin), lambda i: (i, 0)),
                  pl.BlockSpec((fin, fout), lambda i: (0, 0))],
        out_specs=pl.BlockSpec((BLK, 128), lambda i: (i, 0)),
        out_shape=jax.ShapeDtypeStruct((N, 128), jnp.float32),
    )(x, w)


def _layer(spmm, eagg, h_in, We, Ws, Wm_next, bm, be, bs):
    fin, fout = Ws.shape
    fnext = Wm_next.shape[1]
    bm2, be2, bs2 = (b.reshape(1, fout) for b in (bm, be, bs))

    def body(sp_ref, ea_ref, h_ref, we_ref, ws_ref, wmn_ref,
             bm_ref, be_ref, bs_ref, ho_ref, xo_ref):
        sp = (sp_ref[0] + sp_ref[1])[:, :fout]
        ea = ea_ref[0] + ea_ref[1]
        z = (sp
             + jnp.dot(h_ref[...], ws_ref[...], preferred_element_type=jnp.float32)
             + jnp.dot(ea[:, :16], we_ref[...], preferred_element_type=jnp.float32)
             + ea[:, 16:17] * (bm_ref[...] + be_ref[...])
             + bs_ref[...])
        h = _elu(z)
        ho_ref[...] = h
        xo_ref[...] = _pad128(jnp.dot(h, wmn_ref[...],
                                      preferred_element_type=jnp.float32))

    return pl.pallas_call(
        body,
        grid=(GRID,),
        in_specs=[pl.BlockSpec((2, BLK, 128), lambda i: (0, i, 0)),
                  pl.BlockSpec((2, BLK, 128), lambda i: (0, i, 0)),
                  pl.BlockSpec((BLK, fin), lambda i: (i, 0)),
                  pl.BlockSpec((16, fout), lambda i: (0, 0)),
                  pl.BlockSpec((fin, fout), lambda i: (0, 0)),
                  pl.BlockSpec((fout, fnext), lambda i: (0, 0)),
                  pl.BlockSpec((1, fout), lambda i: (0, 0)),
                  pl.BlockSpec((1, fout), lambda i: (0, 0)),
                  pl.BlockSpec((1, fout), lambda i: (0, 0))],
        out_specs=[pl.BlockSpec((BLK, fout), lambda i: (i, 0)),
                   pl.BlockSpec((BLK, 128), lambda i: (i, 0))],
        out_shape=[jax.ShapeDtypeStruct((N, fout), jnp.float32),
                   jax.ShapeDtypeStruct((N, 128), jnp.float32)],
    )(spmm, eagg, h_in, We, Ws, Wm_next, bm2, be2, bs2)


def _final(spmm, eagg, h_in, We, Ws, bm, be, bs, batch3):
    fin, fout = Ws.shape
    bm2, be2, bs2 = (b.reshape(1, fout) for b in (bm, be, bs))

    def body(sp_ref, ea_ref, h_ref, we_ref, ws_ref,
             bm_ref, be_ref, bs_ref, b_ref, o_ref, acc_p, acc_c):
        i = pl.program_id(0)
        sp = (sp_ref[0] + sp_ref[1])[:, :fout]
        ea = ea_ref[0] + ea_ref[1]
        z = (sp
             + jnp.dot(h_ref[...], ws_ref[...], preferred_element_type=jnp.float32)
             + jnp.dot(ea[:, :16], we_ref[...], preferred_element_type=jnp.float32)
             + ea[:, 16:17] * (bm_ref[...] + be_ref[...])
             + bs_ref[...])
        h = _elu(z)
        onehot = (lax.broadcasted_iota(jnp.int32, (G, BLK), 0)
                  == jnp.reshape(b_ref[...], (1, BLK))).astype(jnp.float32)

        @pl.when(i == 0)
        def _():
            acc_p[...] = jnp.zeros((G, fout), jnp.float32)
            acc_c[...] = jnp.zeros((G, fout), jnp.float32)

        acc_p[...] += jnp.dot(onehot, h, preferred_element_type=jnp.float32)
        acc_c[...] += jnp.dot(onehot, jnp.ones((BLK, fout), jnp.float32),
                              preferred_element_type=jnp.float32)

        @pl.when(i == GRID - 1)
        def _():
            o_ref[...] = acc_p[...] / jnp.maximum(acc_c[...], 1.0)

    return pl.pallas_call(
        body,
        grid=(GRID,),
        in_specs=[pl.BlockSpec((2, BLK, 128), lambda i: (0, i, 0)),
                  pl.BlockSpec((2, BLK, 128), lambda i: (0, i, 0)),
                  pl.BlockSpec((BLK, fin), lambda i: (i, 0)),
                  pl.BlockSpec((16, fout), lambda i: (0, 0)),
                  pl.BlockSpec((fin, fout), lambda i: (0, 0)),
                  pl.BlockSpec((1, fout), lambda i: (0, 0)),
                  pl.BlockSpec((1, fout), lambda i: (0, 0)),
                  pl.BlockSpec((1, fout), lambda i: (0, 0)),
                  pl.BlockSpec((1, 1, BLK), lambda i: (i, 0, 0))],
        out_specs=pl.BlockSpec((G, fout), lambda i: (0, 0)),
        out_shape=jax.ShapeDtypeStruct((G, fout), jnp.float32),
        scratch_shapes=[pltpu.VMEM((G, fout), jnp.float32),
                        pltpu.VMEM((G, fout), jnp.float32)],
    )(spmm, eagg, h_in, We, Ws, bm2, be2, bs2, batch3)


def kernel(x, edge_index, edge_attr, batch,
           Wm1, bm1, We1, be1, Ws1, bs1,
           Wm2, bm2, We2, be2, Ws2, bs2,
           Wm3, bm3, We3, be3, Ws3, bs3):
    src = edge_index[0]
    dst = edge_index[1]
    epw_real = E // NW
    padw = EPW - epw_real
    dump = N + ((jnp.arange(padw, dtype=jnp.int32)[None, :]
                 + 7 * jnp.arange(NW, dtype=jnp.int32)[:, None]) % (NP - N))
    src4 = jnp.concatenate(
        [src.reshape(NW, epw_real),
         jnp.zeros((NW, padw), jnp.int32)], axis=1).reshape(NW, CH, 1, CB)
    dst3 = jnp.concatenate(
        [dst.reshape(NW, epw_real), dump], axis=1).reshape(NW, CH, CB)
    ea128 = jnp.concatenate([edge_attr,
                             jnp.ones((E, 1), jnp.float32),
                             jnp.zeros((E, 111), jnp.float32)], axis=1)
    ea4 = jnp.concatenate([ea128.reshape(NW, epw_real, 128),
                           jnp.zeros((NW, padw, 128), jnp.float32)],
                          axis=1).reshape(NW, CH, CB, 128)
    batch3 = batch.reshape(GRID, 1, BLK)
    z128 = jnp.zeros((RPW, 128), jnp.float32)

    xm1 = _proj(x, Wm1)
    eagg = _make_eagg()(ea4, dst3, z128)
    spmm1 = _make_spmm()(xm1, src4, dst3, z128)
    h1, xm2 = _layer(spmm1, eagg, x, We1, Ws1, Wm2, bm1, be1, bs1)
    spmm2 = _make_spmm()(xm2, src4, dst3, z128)
    h2, xm3 = _layer(spmm2, eagg, h1, We2, Ws2, Wm3, bm2, be2, bs2)
    spmm3 = _make_spmm()(xm3, src4, dst3, z128)
    return _final(spmm3, eagg, h2, We3, Ws3, bm3, be3, bs3, batch3)
